# Initial kernel scaffold; baseline (speedup 1.0000x reference)
import jax
import jax.numpy as jnp
from jax.experimental import pallas as pl


def kernel(x, hidden_states, edge_index, params):
    raise NotImplementedError("write your pallas kernel here")


# speedup vs baseline: 6.6155x; 6.6155x over previous
import functools

import jax
import jax.numpy as jnp
from jax import lax
from jax.experimental import pallas as pl
from jax.experimental.pallas import tpu as pltpu
from jax.experimental.pallas import tpu_sc as plsc

_N = 10000
_E = 320000
_D = 128
_L = 2

_NC = 2
_NS = 16
_NW = _NC * _NS
_EPW = _E // _NW
_CHUNK = 80
_NCHUNK = _EPW // _CHUNK
_STRIPE = 624
_LAST_BASE = _STRIPE * (_NS - 1)
_LAST_ROWS = _N - _LAST_BASE


def _make_agg():
    mesh = plsc.VectorSubcoreMesh(core_axis_name="c", subcore_axis_name="s")

    @functools.partial(
        pl.kernel,
        mesh=mesh,
        out_type=[
            jax.ShapeDtypeStruct((_N, _D), jnp.float32),
            jax.ShapeDtypeStruct((_N, _D), jnp.float32),
        ],
        scratch_types=[
            pltpu.VMEM((_NCHUNK, _CHUNK), jnp.int32),
            pltpu.VMEM((_NCHUNK, _CHUNK), jnp.int32),
            pltpu.VMEM((_CHUNK, _D), jnp.float32),
            pltpu.VMEM_SHARED((_N, _D), jnp.float32),
            pltpu.SemaphoreType.DMA,
        ],
    )
    def agg(v_hbm, src_hbm, dst_hbm, zeros_hbm, p0_hbm, p1_hbm,
            src_v, dst_v, rows_v, acc, gsem):
        core = lax.axis_index("c")
        sub = lax.axis_index("s")
        wid = core * _NS + sub
        stripe = pl.multiple_of(sub * _STRIPE, 8)
        is_last = sub == _NS - 1

        @pl.when(jnp.logical_not(is_last))
        def _():
            pltpu.sync_copy(zeros_hbm.at[pl.ds(0, _STRIPE)],
                            acc.at[pl.ds(stripe, _STRIPE)])

        @pl.when(is_last)
        def _():
            pltpu.sync_copy(zeros_hbm, acc.at[pl.ds(_LAST_BASE, _LAST_ROWS)])

        pltpu.sync_copy(src_hbm.at[wid], src_v)
        pltpu.sync_copy(dst_hbm.at[wid], dst_v)
        plsc.subcore_barrier()

        def body(j, carry):
            pltpu.async_copy(v_hbm.at[src_v.at[j]], rows_v, gsem).wait()
            pltpu.sync_copy(rows_v, acc.at[dst_v.at[j]], add=True)
            return carry

        lax.fori_loop(0, _NCHUNK, body, 0)
        plsc.subcore_barrier()

        @pl.when(jnp.logical_and(core == 0, jnp.logical_not(is_last)))
        def _():
            pltpu.sync_copy(acc.at[pl.ds(stripe, _STRIPE)],
                            p0_hbm.at[pl.ds(stripe, _STRIPE)])

        @pl.when(jnp.logical_and(core == 0, is_last))
        def _():
            pltpu.sync_copy(acc.at[pl.ds(_LAST_BASE, _LAST_ROWS)],
                            p0_hbm.at[pl.ds(_LAST_BASE, _LAST_ROWS)])

        @pl.when(jnp.logical_and(core == 1, jnp.logical_not(is_last)))
        def _():
            pltpu.sync_copy(acc.at[pl.ds(stripe, _STRIPE)],
                            p1_hbm.at[pl.ds(stripe, _STRIPE)])

        @pl.when(jnp.logical_and(core == 1, is_last))
        def _():
            pltpu.sync_copy(acc.at[pl.ds(_LAST_BASE, _LAST_ROWS)],
                            p1_hbm.at[pl.ds(_LAST_BASE, _LAST_ROWS)])

    return agg


_agg = _make_agg()

_BLK = 1000


def _tc1_body(ax0, ax1, ah0, ah1, h, wx, wh, b, u_out, hp_out):
    ax = ax0[...] + ax1[...]
    ah = ah0[...] + ah1[...]
    g = jnp.dot(ax, wx[...], preferred_element_type=jnp.float32)
    g = g + jnp.dot(ah, wh[...], preferred_element_type=jnp.float32)
    g = jax.nn.sigmoid(g + b[...])
    u_out[...] = g[:, _D:]
    hp_out[...] = g[:, :_D] * h[...]


def _tc1(ax0, ax1, ah0, ah1, h, wx, wh, b):
    row = pl.BlockSpec((_BLK, _D), lambda i: (i, 0))
    full = pl.BlockSpec((_D, 2 * _D), lambda i: (0, 0))
    bias = pl.BlockSpec((1, 2 * _D), lambda i: (0, 0))
    return pl.pallas_call(
        _tc1_body,
        grid=(_N // _BLK,),
        in_specs=[row, row, row, row, row, full, full, bias],
        out_specs=[row, row],
        out_shape=[
            jax.ShapeDtypeStruct((_N, _D), jnp.float32),
            jax.ShapeDtypeStruct((_N, _D), jnp.float32),
        ],
    )(ax0, ax1, ah0, ah1, h, wx, wh, b)


def _tc2_body(ax0, ax1, ac0, ac1, u, h, wxc, whc, bc, out):
    axv = ax0[...] + ax1[...]
    acv = ac0[...] + ac1[...]
    c = jnp.dot(axv, wxc[...], preferred_element_type=jnp.float32)
    c = c + jnp.dot(acv, whc[...], preferred_element_type=jnp.float32)
    c = jax.nn.sigmoid(c + bc[...])
    uv = u[...]
    out[...] = uv * h[...] + (1.0 - uv) * c


def _tc2(ax0, ax1, ac0, ac1, u, h, wxc, whc, bc):
    row = pl.BlockSpec((_BLK, _D), lambda i: (i, 0))
    full = pl.BlockSpec((_D, _D), lambda i: (0, 0))
    bias = pl.BlockSpec((1, _D), lambda i: (0, 0))
    return pl.pallas_call(
        _tc2_body,
        grid=(_N // _BLK,),
        in_specs=[row, row, row, row, row, row, full, full, bias],
        out_specs=row,
        out_shape=jax.ShapeDtypeStruct((_N, _D), jnp.float32),
    )(ax0, ax1, ac0, ac1, u, h, wxc, whc, bc)


def kernel(x, hidden_states, edge_index, params):
    src_rs = edge_index[0].reshape(_NW, _NCHUNK, _CHUNK)
    dst_rs = edge_index[1].reshape(_NW, _NCHUNK, _CHUNK)
    zeros = jnp.zeros((_LAST_ROWS, _D), jnp.float32)

    hiddens = []
    cur = x
    for l in range(_L):
        h = hidden_states[l]
        wr, wu, wc = params['W_r%d' % l], params['W_u%d' % l], params['W_c%d' % l]
        wx_ru = jnp.concatenate([wr[:_D], wu[:_D]], axis=1)
        wh_ru = jnp.concatenate([wr[_D:], wu[_D:]], axis=1)
        b_ru = jnp.concatenate([
            params['b_r%d' % l] + params['gb_r%d' % l],
            params['b_u%d' % l] + params['gb_u%d' % l],
        ])[None, :]
        wxc, whc = wc[:_D], wc[_D:]
        bc = (params['b_c%d' % l] + params['gb_c%d' % l])[None, :]

        ax0, ax1 = _agg(cur, src_rs, dst_rs, zeros)
        ah0, ah1 = _agg(h, src_rs, dst_rs, zeros)
        u, hp = _tc1(ax0, ax1, ah0, ah1, h, wx_ru, wh_ru, b_ru)
        ac0, ac1 = _agg(hp, src_rs, dst_rs, zeros)
        cur = _tc2(ax0, ax1, ac0, ac1, u, h, wxc, whc, bc)
        hiddens.append(cur)
    return (cur, jnp.stack(hiddens))


# speedup vs baseline: 8.5834x; 1.2975x over previous
import functools

import jax
import jax.numpy as jnp
from jax import lax
from jax.experimental import pallas as pl
from jax.experimental.pallas import tpu as pltpu
from jax.experimental.pallas import tpu_sc as plsc

_N = 10000
_E = 320000
_D = 128
_L = 2

_NC = 2
_NS = 16
_NW = _NC * _NS
_EPW = _E // _NW
_CHUNK = 80
_NCHUNK = _EPW // _CHUNK
_STRIPE = 624
_LAST_BASE = _STRIPE * (_NS - 1)
_LAST_ROWS = _N - _LAST_BASE


def _make_agg():
    mesh = plsc.VectorSubcoreMesh(core_axis_name="c", subcore_axis_name="s")

    @functools.partial(
        pl.kernel,
        mesh=mesh,
        out_type=[
            jax.ShapeDtypeStruct((_N, _D), jnp.float32),
            jax.ShapeDtypeStruct((_N, _D), jnp.float32),
        ],
        scratch_types=[
            pltpu.VMEM((_EPW,), jnp.int32),
            pltpu.VMEM((_NCHUNK, _CHUNK), jnp.int32),
            pltpu.VMEM((_CHUNK, _D), jnp.float32),
            pltpu.VMEM((_CHUNK, _D), jnp.float32),
            pltpu.VMEM_SHARED((_N, _D), jnp.float32),
            pltpu.SemaphoreType.DMA,
            pltpu.SemaphoreType.DMA,
        ],
    )
    def agg(v_hbm, src_hbm, dst_hbm, zeros_hbm, p0_hbm, p1_hbm,
            src_v, dst_v, buf_a, buf_b, acc, gsem_a, gsem_b):
        core = lax.axis_index("c")
        sub = lax.axis_index("s")
        wid = core * _NS + sub
        stripe = pl.multiple_of(sub * _STRIPE, 8)
        is_last = sub == _NS - 1

        @pl.when(jnp.logical_not(is_last))
        def _():
            pltpu.sync_copy(zeros_hbm.at[pl.ds(0, _STRIPE)],
                            acc.at[pl.ds(stripe, _STRIPE)])

        @pl.when(is_last)
        def _():
            pltpu.sync_copy(zeros_hbm, acc.at[pl.ds(_LAST_BASE, _LAST_ROWS)])

        pltpu.sync_copy(src_hbm.at[wid], src_v)
        pltpu.sync_copy(dst_hbm.at[wid], dst_v)
        plsc.subcore_barrier()

        def src_at(j):
            return src_v.at[pl.ds(pl.multiple_of(j * _CHUNK, 8), _CHUNK)]

        pltpu.async_copy(v_hbm.at[src_at(0)], buf_a, gsem_a)

        def body(i, carry):
            j0 = i * 2
            j1 = j0 + 1
            pltpu.make_async_copy(v_hbm.at[src_at(j0)], buf_a, gsem_a).wait()
            cp = pltpu.async_copy(v_hbm.at[src_at(j1)], buf_b, gsem_b)
            pltpu.sync_copy(buf_a, acc.at[dst_v.at[j0]], add=True)
            cp.wait()

            @pl.when(j1 + 1 < _NCHUNK)
            def _():
                pltpu.async_copy(v_hbm.at[src_at(j1 + 1)], buf_a, gsem_a)

            pltpu.sync_copy(buf_b, acc.at[dst_v.at[j1]], add=True)
            return carry

        lax.fori_loop(0, _NCHUNK // 2, body, 0)
        if _NCHUNK % 2:
            last = _NCHUNK - 1
            pltpu.make_async_copy(v_hbm.at[src_at(last)], buf_a, gsem_a).wait()
            pltpu.sync_copy(buf_a, acc.at[dst_v.at[last]], add=True)
        plsc.subcore_barrier()

        @pl.when(jnp.logical_and(core == 0, jnp.logical_not(is_last)))
        def _():
            pltpu.sync_copy(acc.at[pl.ds(stripe, _STRIPE)],
                            p0_hbm.at[pl.ds(stripe, _STRIPE)])

        @pl.when(jnp.logical_and(core == 0, is_last))
        def _():
            pltpu.sync_copy(acc.at[pl.ds(_LAST_BASE, _LAST_ROWS)],
                            p0_hbm.at[pl.ds(_LAST_BASE, _LAST_ROWS)])

        @pl.when(jnp.logical_and(core == 1, jnp.logical_not(is_last)))
        def _():
            pltpu.sync_copy(acc.at[pl.ds(stripe, _STRIPE)],
                            p1_hbm.at[pl.ds(stripe, _STRIPE)])

        @pl.when(jnp.logical_and(core == 1, is_last))
        def _():
            pltpu.sync_copy(acc.at[pl.ds(_LAST_BASE, _LAST_ROWS)],
                            p1_hbm.at[pl.ds(_LAST_BASE, _LAST_ROWS)])

    return agg


_agg = _make_agg()

_BLK = 1000


def _tc1_body(ax0, ax1, ah0, ah1, h, wx, wh, b, u_out, hp_out):
    ax = ax0[...] + ax1[...]
    ah = ah0[...] + ah1[...]
    g = jnp.dot(ax, wx[...], preferred_element_type=jnp.float32)
    g = g + jnp.dot(ah, wh[...], preferred_element_type=jnp.float32)
    g = jax.nn.sigmoid(g + b[...])
    u_out[...] = g[:, _D:]
    hp_out[...] = g[:, :_D] * h[...]


def _tc1(ax0, ax1, ah0, ah1, h, wx, wh, b):
    row = pl.BlockSpec((_BLK, _D), lambda i: (i, 0))
    full = pl.BlockSpec((_D, 2 * _D), lambda i: (0, 0))
    bias = pl.BlockSpec((1, 2 * _D), lambda i: (0, 0))
    return pl.pallas_call(
        _tc1_body,
        grid=(_N // _BLK,),
        in_specs=[row, row, row, row, row, full, full, bias],
        out_specs=[row, row],
        out_shape=[
            jax.ShapeDtypeStruct((_N, _D), jnp.float32),
            jax.ShapeDtypeStruct((_N, _D), jnp.float32),
        ],
    )(ax0, ax1, ah0, ah1, h, wx, wh, b)


def _tc2_body(ax0, ax1, ac0, ac1, u, h, wxc, whc, bc, out):
    axv = ax0[...] + ax1[...]
    acv = ac0[...] + ac1[...]
    c = jnp.dot(axv, wxc[...], preferred_element_type=jnp.float32)
    c = c + jnp.dot(acv, whc[...], preferred_element_type=jnp.float32)
    c = jax.nn.sigmoid(c + bc[...])
    uv = u[...]
    out[...] = uv * h[...] + (1.0 - uv) * c


def _tc2(ax0, ax1, ac0, ac1, u, h, wxc, whc, bc):
    row = pl.BlockSpec((_BLK, _D), lambda i: (i, 0))
    full = pl.BlockSpec((_D, _D), lambda i: (0, 0))
    bias = pl.BlockSpec((1, _D), lambda i: (0, 0))
    return pl.pallas_call(
        _tc2_body,
        grid=(_N // _BLK,),
        in_specs=[row, row, row, row, row, row, full, full, bias],
        out_specs=row,
        out_shape=jax.ShapeDtypeStruct((_N, _D), jnp.float32),
    )(ax0, ax1, ac0, ac1, u, h, wxc, whc, bc)


def kernel(x, hidden_states, edge_index, params):
    src_rs = edge_index[0].reshape(_NW, _EPW)
    dst_rs = edge_index[1].reshape(_NW, _NCHUNK, _CHUNK)
    zeros = jnp.zeros((_LAST_ROWS, _D), jnp.float32)

    hiddens = []
    cur = x
    for l in range(_L):
        h = hidden_states[l]
        wr, wu, wc = params['W_r%d' % l], params['W_u%d' % l], params['W_c%d' % l]
        wx_ru = jnp.concatenate([wr[:_D], wu[:_D]], axis=1)
        wh_ru = jnp.concatenate([wr[_D:], wu[_D:]], axis=1)
        b_ru = jnp.concatenate([
            params['b_r%d' % l] + params['gb_r%d' % l],
            params['b_u%d' % l] + params['gb_u%d' % l],
        ])[None, :]
        wxc, whc = wc[:_D], wc[_D:]
        bc = (params['b_c%d' % l] + params['gb_c%d' % l])[None, :]

        ax0, ax1 = _agg(cur, src_rs, dst_rs, zeros)
        ah0, ah1 = _agg(h, src_rs, dst_rs, zeros)
        u, hp = _tc1(ax0, ax1, ah0, ah1, h, wx_ru, wh_ru, b_ru)
        ac0, ac1 = _agg(hp, src_rs, dst_rs, zeros)
        cur = _tc2(ax0, ax1, ac0, ac1, u, h, wxc, whc, bc)
        hiddens.append(cur)
    return (cur, jnp.stack(hiddens))
